# baseline (speedup 1.0000x reference)
import jax
import jax.numpy as jnp
from jax import lax
from jax.experimental import pallas as pl
from jax.experimental.pallas import tpu as pltpu
from jax.experimental.pallas import tpu_sc as plsc

H = 128
HH = 64
NJ = 50000
E = 500000
EP = 524288
ROW = 128
RPS = EP // (16 * ROW)
SB = 32
NSB = RPS // SB
NBUF = 4

_RELS = [(20000, 20096), (20000, 20096), (20000, 20096),
         (20000, 20096), (10000, 10112)]
_ACC_ROWS = 20096


def _sc_counts_body(dst0, dst1, dst2, dst3, dst4,
                    ones_h, z16,
                    C0, C1, C2, C3, C4,
                    didx, ones, cacc0, cacc1, cacc2, cacc3,
                    c0s, c1s, c2s, c3s):
    c = lax.axis_index("c")
    s = lax.axis_index("s")
    dsts = [dst0, dst1, dst2, dst3, dst4]
    Cs = [C0, C1, C2, C3, C4]
    csems = [c0s, c1s, c2s, c3s]
    caccs = [cacc0, cacc1, cacc2, cacc3]
    pltpu.sync_copy(ones_h, ones)
    for r, (ndst, np_) in enumerate(_RELS):
        nz = np_ // 16
        for m in range(4):
            pltpu.sync_copy(z16.at[pl.ds(0, nz)],
                            caccs[m].at[pl.ds(s * nz, nz)])
        plsc.subcore_barrier()
        dst2d = dsts[r]

        @pl.loop(4 * c, 4 * c + 4)
        def _(b):
            pltpu.sync_copy(dst2d.at[pl.ds(s * RPS + b * SB, SB)],
                            didx.at[0])

            @pl.loop(0, SB // 4)
            def _(p):
                for m in range(4):
                    j = 4 * p + m

                    @pl.when(p > 0)
                    def _():
                        pltpu.make_async_copy(
                            ones, caccs[m].at[didx.at[0, 0]],
                            csems[m]).wait()

                    pltpu.async_copy(ones, caccs[m].at[didx.at[0, j]],
                                     csems[m], add=True)

            for m in range(4):
                pltpu.make_async_copy(
                    ones, caccs[m].at[didx.at[0, 0]], csems[m]).wait()

        plsc.subcore_barrier()
        for m in range(4):
            pltpu.sync_copy(caccs[m].at[pl.ds(s * nz, nz), pl.ds(0, 8)],
                            Cs[r].at[4 * c + m, pl.ds(s * nz, nz)])


def _sc_rows_body(x0h, x1h,
                  src0, dst0, src1, dst1, src2, dst2, src3, dst3, src4, dst4,
                  z64,
                  S0, S1, S2, S3, S4,
                  sidx, didx, r0, r1, r2, r3, acc,
                  isem0, isem1,
                  g0, g1, g2, g3, s0, s1, s2, s3):
    c = lax.axis_index("c")
    s = lax.axis_index("s")
    srcs = [src0, src1, src2, src3, src4]
    dsts = [dst0, dst1, dst2, dst3, dst4]
    Ss = [S0, S1, S2, S3, S4]
    rbufs = [r0, r1, r2, r3]
    gsems = [g0, g1, g2, g3]
    ssems = [s0, s1, s2, s3]
    isems = [isem0, isem1]

    for r, (ndst, np_) in enumerate(_RELS):
        nz = np_ // 16
        pltpu.sync_copy(z64.at[pl.ds(0, nz)], acc.at[pl.ds(s * nz, nz)])
        plsc.subcore_barrier()
        src2d, dst2d = srcs[r], dsts[r]

        def fire_stage(b, q):
            pltpu.async_copy(src2d.at[pl.ds(s * RPS + b * SB, SB)],
                             sidx.at[q], isems[q])
            pltpu.async_copy(dst2d.at[pl.ds(s * RPS + b * SB, SB)],
                             didx.at[q], isems[q])

        def wait_stage(q):
            pltpu.make_async_copy(src2d.at[pl.ds(s * RPS, SB)],
                                  sidx.at[q], isems[q]).wait()
            pltpu.make_async_copy(dst2d.at[pl.ds(s * RPS, SB)],
                                  didx.at[q], isems[q]).wait()

        def fire_gather(q, jrow, rbuf, sem):
            @pl.when(c == 0)
            def _():
                pltpu.async_copy(x0h.at[sidx.at[q, jrow]], rbuf, sem)

            @pl.when(c == 1)
            def _():
                pltpu.async_copy(x1h.at[sidx.at[q, jrow]], rbuf, sem)

        def wait_gather(rbuf, sem):
            pltpu.make_async_copy(x0h.at[sidx.at[0, 0]], rbuf, sem).wait()

        def fire_scatter(q, jrow, rbuf, sem):
            pltpu.async_copy(rbuf, acc.at[didx.at[q, jrow]], sem, add=True)

        def wait_scatter(rbuf, sem):
            pltpu.make_async_copy(rbuf, acc.at[didx.at[0, 0]], sem).wait()

        def run_rows(q):
            for k in range(NBUF - 1):
                fire_gather(q, k, rbufs[k], gsems[k])

            @pl.loop(0, SB // NBUF)
            def _(p):
                for t in range(NBUF):
                    j = NBUF * p + t
                    wait_gather(rbufs[t], gsems[t])
                    if t == 0:
                        @pl.when(p > 0)
                        def _():
                            wait_scatter(rbufs[t], ssems[0])
                    else:
                        wait_scatter(rbufs[t], ssems[0])
                    fire_scatter(q, j, rbufs[t], ssems[0])
                    nt = (t + NBUF - 1) % NBUF
                    lim = (SB - 1 - t - (NBUF - 1)) // NBUF

                    def adv():
                        fire_gather(q, j + NBUF - 1, rbufs[nt], gsems[nt])

                    if lim >= SB // NBUF - 1:
                        adv()
                    else:
                        @pl.when(p <= lim)
                        def _():
                            adv()

            wait_scatter(rbufs[0], ssems[0])

        fire_stage(0, 0)
        wait_stage(0)

        @pl.loop(0, NSB)
        def _(b):
            q = lax.rem(b, 2)

            @pl.when((b > 0) & (q == 0))
            def _():
                wait_stage(0)

            @pl.when((b > 0) & (q != 0))
            def _():
                wait_stage(1)

            @pl.when((b + 1 < NSB) & (q == 0))
            def _():
                fire_stage(b + 1, 1)

            @pl.when((b + 1 < NSB) & (q != 0))
            def _():
                fire_stage(b + 1, 0)

            run_rows(q)

        plsc.subcore_barrier()
        pltpu.sync_copy(acc.at[pl.ds(s * nz, nz)],
                        Ss[r].at[c, pl.ds(s * nz, nz)])


def _sc_segsums(x0h, x1h, eis):
    mesh = plsc.VectorSubcoreMesh(core_axis_name="c", subcore_axis_name="s")
    zrows = _ACC_ROWS // 16
    cnt_out = [jax.ShapeDtypeStruct((8, np_, 8), jnp.float32)
               for _, np_ in _RELS]
    row_out = [jax.ShapeDtypeStruct((2, np_, HH), jnp.float32)
               for _, np_ in _RELS]
    cnt_fn = pl.kernel(
        _sc_counts_body,
        out_type=cnt_out,
        mesh=mesh,
        scratch_types=[
            pltpu.VMEM((1, SB, ROW), jnp.int32),
            pltpu.VMEM((ROW, 16), jnp.float32),
            pltpu.VMEM_SHARED((_ACC_ROWS, 16), jnp.float32),
            pltpu.VMEM_SHARED((_ACC_ROWS, 16), jnp.float32),
            pltpu.VMEM_SHARED((_ACC_ROWS, 16), jnp.float32),
            pltpu.VMEM_SHARED((_ACC_ROWS, 16), jnp.float32),
            pltpu.SemaphoreType.DMA,
            pltpu.SemaphoreType.DMA,
            pltpu.SemaphoreType.DMA,
            pltpu.SemaphoreType.DMA,
        ],
        compiler_params=pltpu.CompilerParams(use_tc_tiling_on_sc=False),
    )
    row_fn = pl.kernel(
        _sc_rows_body,
        out_type=row_out,
        mesh=mesh,
        scratch_types=[
            pltpu.VMEM((2, SB, ROW), jnp.int32),
            pltpu.VMEM((2, SB, ROW), jnp.int32),
            pltpu.VMEM((ROW, HH), jnp.float32),
            pltpu.VMEM((ROW, HH), jnp.float32),
            pltpu.VMEM((ROW, HH), jnp.float32),
            pltpu.VMEM((ROW, HH), jnp.float32),
            pltpu.VMEM_SHARED((_ACC_ROWS, HH), jnp.float32),
            pltpu.SemaphoreType.DMA,
            pltpu.SemaphoreType.DMA,
            pltpu.SemaphoreType.DMA,
            pltpu.SemaphoreType.DMA,
            pltpu.SemaphoreType.DMA,
            pltpu.SemaphoreType.DMA,
            pltpu.SemaphoreType.DMA,
            pltpu.SemaphoreType.DMA,
            pltpu.SemaphoreType.DMA,
            pltpu.SemaphoreType.DMA,
        ],
        compiler_params=pltpu.CompilerParams(use_tc_tiling_on_sc=False),
    )
    ones_h = jnp.ones((ROW, 16), jnp.float32)
    z64 = jnp.zeros((zrows, HH), jnp.float32)
    z16 = jnp.zeros((zrows, 16), jnp.float32)
    dst_flat = [dstp for (_, dstp) in eis]
    sd_flat = []
    for (srcp, dstp) in eis:
        sd_flat.append(srcp)
        sd_flat.append(dstp)
    Cs = cnt_fn(*dst_flat, ones_h, z16)
    Ss = row_fn(x0h, x1h, *sd_flat, z64)
    return Ss, Cs


def _dense2_body(S0, C0, S1, C1, x, W0T, W1T, WrT, b, o):
    c0 = jnp.maximum(jnp.sum(C0[:, :, 0:1], axis=0), 1.0)
    c1 = jnp.maximum(jnp.sum(C1[:, :, 0:1], axis=0), 1.0)
    m0 = jnp.concatenate([S0[0], S0[1]], axis=1) / c0
    m1 = jnp.concatenate([S1[0], S1[1]], axis=1) / c1
    acc = (jnp.dot(m0, W0T[...], preferred_element_type=jnp.float32,
                   precision=lax.Precision.HIGHEST)
           + jnp.dot(m1, W1T[...], preferred_element_type=jnp.float32,
                     precision=lax.Precision.HIGHEST)
           + jnp.dot(x[...], WrT[...], preferred_element_type=jnp.float32,
                     precision=lax.Precision.HIGHEST)
           + b[...])
    o[...] = jnp.maximum(acc, 0.0)


def _dense1_body(S0, C0, x, W0T, WrT, b, o):
    c0 = jnp.maximum(jnp.sum(C0[:, :, 0:1], axis=0), 1.0)
    m0 = jnp.concatenate([S0[0], S0[1]], axis=1) / c0
    acc = (jnp.dot(m0, W0T[...], preferred_element_type=jnp.float32,
                   precision=lax.Precision.HIGHEST)
           + jnp.dot(x[...], WrT[...], preferred_element_type=jnp.float32,
                     precision=lax.Precision.HIGHEST)
           + b[...])
    o[...] = jnp.maximum(acc, 0.0)


_BLK = 2000


def _dense2(S0, C0, S1, C1, x, W0T, W1T, WrT, b):
    n = x.shape[0]
    grid = (n // _BLK,)
    sspec = pl.BlockSpec((2, _BLK, HH), lambda i: (0, i, 0))
    cspec = pl.BlockSpec((8, _BLK, 8), lambda i: (0, i, 0))
    wspec = pl.BlockSpec((H, H), lambda i: (0, 0))
    bspec = pl.BlockSpec((1, H), lambda i: (0, 0))
    xspec = pl.BlockSpec((_BLK, H), lambda i: (i, 0))
    return pl.pallas_call(
        _dense2_body,
        grid=grid,
        in_specs=[sspec, cspec, sspec, cspec, xspec, wspec, wspec, wspec, bspec],
        out_specs=xspec,
        out_shape=jax.ShapeDtypeStruct((n, H), jnp.float32),
    )(S0, C0, S1, C1, x, W0T, W1T, WrT, b)


def _dense1(S0, C0, x, W0T, WrT, b):
    n = x.shape[0]
    grid = (n // _BLK,)
    sspec = pl.BlockSpec((2, _BLK, HH), lambda i: (0, i, 0))
    cspec = pl.BlockSpec((8, _BLK, 8), lambda i: (0, i, 0))
    wspec = pl.BlockSpec((H, H), lambda i: (0, 0))
    bspec = pl.BlockSpec((1, H), lambda i: (0, 0))
    xspec = pl.BlockSpec((_BLK, H), lambda i: (i, 0))
    return pl.pallas_call(
        _dense1_body,
        grid=grid,
        in_specs=[sspec, cspec, xspec, wspec, wspec, bspec],
        out_specs=xspec,
        out_shape=jax.ShapeDtypeStruct((n, H), jnp.float32),
    )(S0, C0, x, W0T, WrT, b)


def _pad_edges(ei, ndst, np_):
    pad_n = EP - E
    ar = jnp.arange(pad_n, dtype=jnp.int32)
    pad_src = ar % NJ
    pad_dst = ndst + (ar % (np_ - ndst))
    srcp = jnp.concatenate([ei[0], pad_src]).reshape(EP // ROW, ROW)
    dstp = jnp.concatenate([ei[1], pad_dst]).reshape(EP // ROW, ROW)
    return srcp, dstp


def kernel(x_job, x_station, x_machine, x_robot, ei_cbl, ei_li, ei_nd, ei_eb,
           ei_hb, Wl_cbl, bl_cbl, Wr_cbl, Wl_li, bl_li, Wr_li, Wl_nd, bl_nd,
           Wr_nd, Wl_eb, bl_eb, Wr_eb, Wl_hb, bl_hb, Wr_hb):
    x0h = x_job[:, :HH]
    x1h = x_job[:, HH:]
    eis = [_pad_edges(ei, ndst, np_)
           for ei, (ndst, np_) in zip([ei_cbl, ei_li, ei_nd, ei_eb, ei_hb],
                                      _RELS)]
    (S0, S1, S2, S3, S4), (C0, C1, C2, C3, C4) = _sc_segsums(x0h, x1h, eis)

    h_s = _dense2(S0, C0, S1, C1, x_station,
                  Wl_cbl.T, Wl_li.T, (Wr_cbl + Wr_li).T,
                  (bl_cbl + bl_li)[None, :])
    h_m = _dense2(S2, C2, S3, C3, x_machine,
                  Wl_nd.T, Wl_eb.T, (Wr_nd + Wr_eb).T,
                  (bl_nd + bl_eb)[None, :])
    h_r = _dense1(S4, C4, x_robot, Wl_hb.T, Wr_hb.T, bl_hb[None, :])
    return (h_s, h_m, h_r)


# speedup vs baseline: 1.1033x; 1.1033x over previous
import jax
import jax.numpy as jnp
from jax import lax
from jax.experimental import pallas as pl
from jax.experimental.pallas import tpu as pltpu
from jax.experimental.pallas import tpu_sc as plsc

H = 128
HH = 64
NJ = 50000
E = 500000
EP = 524288
ROW = 128
RPS = EP // (16 * ROW)
SB = 32
NSB = RPS // SB
NBUF = 4

_RELS = [(20000, 20096), (20000, 20096), (20000, 20096),
         (20000, 20096), (10000, 10112)]
_ACC_ROWS = 20096


def _sc_counts_body(dst0, dst1, dst2, dst3, dst4,
                    ones_h, z16,
                    C0, C1, C2, C3, C4,
                    didx, ones, cacc0, cacc1, cacc2, cacc3,
                    c0s, c1s, c2s, c3s):
    c = lax.axis_index("c")
    s = lax.axis_index("s")
    dsts = [dst0, dst1, dst2, dst3, dst4]
    Cs = [C0, C1, C2, C3, C4]
    csems = [c0s, c1s, c2s, c3s]
    caccs = [cacc0, cacc1, cacc2, cacc3]
    pltpu.sync_copy(ones_h, ones)
    for r, (ndst, np_) in enumerate(_RELS):
        nz = np_ // 16
        for m in range(4):
            pltpu.sync_copy(z16.at[pl.ds(0, nz)],
                            caccs[m].at[pl.ds(s * nz, nz)])
        plsc.subcore_barrier()
        dst2d = dsts[r]

        @pl.loop(4 * c, 4 * c + 4)
        def _(b):
            pltpu.sync_copy(dst2d.at[pl.ds(s * RPS + b * SB, SB)],
                            didx.at[0])

            @pl.loop(0, SB // 4)
            def _(p):
                for m in range(4):
                    j = 4 * p + m
                    pltpu.async_copy(ones, caccs[m].at[didx.at[0, j]],
                                     csems[m], add=True)
                    pltpu.make_async_copy(
                        ones, caccs[m].at[didx.at[0, 0]],
                        csems[m]).wait()

        plsc.subcore_barrier()
        for m in range(4):
            pltpu.sync_copy(caccs[m].at[pl.ds(s * nz, nz)],
                            Cs[r].at[4 * c + m, pl.ds(s * nz, nz)])


def _sc_rows_body(x0h, x1h,
                  src0, dst0, src1, dst1, src2, dst2, src3, dst3, src4, dst4,
                  z64,
                  S0, S1, S2, S3, S4,
                  sidx, didx, r0, r1, r2, r3, acc,
                  isem0, isem1,
                  g0, g1, g2, g3, s0, s1, s2, s3):
    c = lax.axis_index("c")
    s = lax.axis_index("s")
    srcs = [src0, src1, src2, src3, src4]
    dsts = [dst0, dst1, dst2, dst3, dst4]
    Ss = [S0, S1, S2, S3, S4]
    rbufs = [r0, r1, r2, r3]
    gsems = [g0, g1, g2, g3]
    ssems = [s0, s1, s2, s3]
    isems = [isem0, isem1]

    for r, (ndst, np_) in enumerate(_RELS):
        nz = np_ // 16
        pltpu.sync_copy(z64.at[pl.ds(0, nz)], acc.at[pl.ds(s * nz, nz)])
        plsc.subcore_barrier()
        src2d, dst2d = srcs[r], dsts[r]

        def fire_stage(b, q):
            pltpu.async_copy(src2d.at[pl.ds(s * RPS + b * SB, SB)],
                             sidx.at[q], isems[q])
            pltpu.async_copy(dst2d.at[pl.ds(s * RPS + b * SB, SB)],
                             didx.at[q], isems[q])

        def wait_stage(q):
            pltpu.make_async_copy(src2d.at[pl.ds(s * RPS, SB)],
                                  sidx.at[q], isems[q]).wait()
            pltpu.make_async_copy(dst2d.at[pl.ds(s * RPS, SB)],
                                  didx.at[q], isems[q]).wait()

        def fire_gather(q, jrow, rbuf, sem):
            @pl.when(c == 0)
            def _():
                pltpu.async_copy(x0h.at[sidx.at[q, jrow]], rbuf, sem)

            @pl.when(c == 1)
            def _():
                pltpu.async_copy(x1h.at[sidx.at[q, jrow]], rbuf, sem)

        def wait_gather(rbuf, sem):
            pltpu.make_async_copy(x0h.at[sidx.at[0, 0]], rbuf, sem).wait()

        def fire_scatter(q, jrow, rbuf, sem):
            pltpu.async_copy(rbuf, acc.at[didx.at[q, jrow]], sem, add=True)

        def wait_scatter(rbuf, sem):
            pltpu.make_async_copy(rbuf, acc.at[didx.at[0, 0]], sem).wait()

        def run_rows(q):
            for k in range(NBUF - 1):
                fire_gather(q, k, rbufs[k], gsems[k])

            @pl.loop(0, SB // NBUF)
            def _(p):
                for t in range(NBUF):
                    j = NBUF * p + t
                    wait_gather(rbufs[t], gsems[t])
                    if t == 0:
                        @pl.when(p > 0)
                        def _():
                            wait_scatter(rbufs[t], ssems[0])
                    else:
                        wait_scatter(rbufs[t], ssems[0])
                    fire_scatter(q, j, rbufs[t], ssems[0])
                    nt = (t + NBUF - 1) % NBUF
                    lim = (SB - 1 - t - (NBUF - 1)) // NBUF

                    def adv():
                        fire_gather(q, j + NBUF - 1, rbufs[nt], gsems[nt])

                    if lim >= SB // NBUF - 1:
                        adv()
                    else:
                        @pl.when(p <= lim)
                        def _():
                            adv()

            wait_scatter(rbufs[0], ssems[0])

        fire_stage(0, 0)
        wait_stage(0)

        @pl.loop(0, NSB)
        def _(b):
            q = lax.rem(b, 2)

            @pl.when((b > 0) & (q == 0))
            def _():
                wait_stage(0)

            @pl.when((b > 0) & (q != 0))
            def _():
                wait_stage(1)

            @pl.when((b + 1 < NSB) & (q == 0))
            def _():
                fire_stage(b + 1, 1)

            @pl.when((b + 1 < NSB) & (q != 0))
            def _():
                fire_stage(b + 1, 0)

            run_rows(q)

        plsc.subcore_barrier()
        pltpu.sync_copy(acc.at[pl.ds(s * nz, nz)],
                        Ss[r].at[c, pl.ds(s * nz, nz)])


def _sc_segsums(x0h, x1h, eis):
    mesh = plsc.VectorSubcoreMesh(core_axis_name="c", subcore_axis_name="s")
    zrows = _ACC_ROWS // 16
    cnt_out = [jax.ShapeDtypeStruct((8, np_, 16), jnp.float32)
               for _, np_ in _RELS]
    row_out = [jax.ShapeDtypeStruct((2, np_, HH), jnp.float32)
               for _, np_ in _RELS]
    cnt_fn = pl.kernel(
        _sc_counts_body,
        out_type=cnt_out,
        mesh=mesh,
        scratch_types=[
            pltpu.VMEM((1, SB, ROW), jnp.int32),
            pltpu.VMEM((ROW, 16), jnp.float32),
            pltpu.VMEM_SHARED((_ACC_ROWS, 16), jnp.float32),
            pltpu.VMEM_SHARED((_ACC_ROWS, 16), jnp.float32),
            pltpu.VMEM_SHARED((_ACC_ROWS, 16), jnp.float32),
            pltpu.VMEM_SHARED((_ACC_ROWS, 16), jnp.float32),
            pltpu.SemaphoreType.DMA,
            pltpu.SemaphoreType.DMA,
            pltpu.SemaphoreType.DMA,
            pltpu.SemaphoreType.DMA,
        ],
        compiler_params=pltpu.CompilerParams(use_tc_tiling_on_sc=False),
    )
    row_fn = pl.kernel(
        _sc_rows_body,
        out_type=row_out,
        mesh=mesh,
        scratch_types=[
            pltpu.VMEM((2, SB, ROW), jnp.int32),
            pltpu.VMEM((2, SB, ROW), jnp.int32),
            pltpu.VMEM((ROW, HH), jnp.float32),
            pltpu.VMEM((ROW, HH), jnp.float32),
            pltpu.VMEM((ROW, HH), jnp.float32),
            pltpu.VMEM((ROW, HH), jnp.float32),
            pltpu.VMEM_SHARED((_ACC_ROWS, HH), jnp.float32),
            pltpu.SemaphoreType.DMA,
            pltpu.SemaphoreType.DMA,
            pltpu.SemaphoreType.DMA,
            pltpu.SemaphoreType.DMA,
            pltpu.SemaphoreType.DMA,
            pltpu.SemaphoreType.DMA,
            pltpu.SemaphoreType.DMA,
            pltpu.SemaphoreType.DMA,
            pltpu.SemaphoreType.DMA,
            pltpu.SemaphoreType.DMA,
        ],
        compiler_params=pltpu.CompilerParams(use_tc_tiling_on_sc=False),
    )
    ones_h = jnp.ones((ROW, 16), jnp.float32)
    z64 = jnp.zeros((zrows, HH), jnp.float32)
    z16 = jnp.zeros((zrows, 16), jnp.float32)
    dst_flat = [dstp for (_, dstp) in eis]
    sd_flat = []
    for (srcp, dstp) in eis:
        sd_flat.append(srcp)
        sd_flat.append(dstp)
    Cs = cnt_fn(*dst_flat, ones_h, z16)
    Ss = row_fn(x0h, x1h, *sd_flat, z64)
    return Ss, Cs


def _dense2_body(S0, C0, S1, C1, x, W0T, W1T, WrT, b, o):
    c0 = jnp.maximum(jnp.sum(C0[:, :, 0:1], axis=0), 1.0)
    c1 = jnp.maximum(jnp.sum(C1[:, :, 0:1], axis=0), 1.0)
    m0 = jnp.concatenate([S0[0], S0[1]], axis=1) / c0
    m1 = jnp.concatenate([S1[0], S1[1]], axis=1) / c1
    acc = (jnp.dot(m0, W0T[...], preferred_element_type=jnp.float32,
                   precision=lax.Precision.HIGHEST)
           + jnp.dot(m1, W1T[...], preferred_element_type=jnp.float32,
                     precision=lax.Precision.HIGHEST)
           + jnp.dot(x[...], WrT[...], preferred_element_type=jnp.float32,
                     precision=lax.Precision.HIGHEST)
           + b[...])
    o[...] = jnp.maximum(acc, 0.0)


def _dense1_body(S0, C0, x, W0T, WrT, b, o):
    c0 = jnp.maximum(jnp.sum(C0[:, :, 0:1], axis=0), 1.0)
    m0 = jnp.concatenate([S0[0], S0[1]], axis=1) / c0
    acc = (jnp.dot(m0, W0T[...], preferred_element_type=jnp.float32,
                   precision=lax.Precision.HIGHEST)
           + jnp.dot(x[...], WrT[...], preferred_element_type=jnp.float32,
                     precision=lax.Precision.HIGHEST)
           + b[...])
    o[...] = jnp.maximum(acc, 0.0)


_BLK = 2000


def _dense2(S0, C0, S1, C1, x, W0T, W1T, WrT, b):
    n = x.shape[0]
    grid = (n // _BLK,)
    sspec = pl.BlockSpec((2, _BLK, HH), lambda i: (0, i, 0))
    cspec = pl.BlockSpec((8, _BLK, 16), lambda i: (0, i, 0))
    wspec = pl.BlockSpec((H, H), lambda i: (0, 0))
    bspec = pl.BlockSpec((1, H), lambda i: (0, 0))
    xspec = pl.BlockSpec((_BLK, H), lambda i: (i, 0))
    return pl.pallas_call(
        _dense2_body,
        grid=grid,
        in_specs=[sspec, cspec, sspec, cspec, xspec, wspec, wspec, wspec, bspec],
        out_specs=xspec,
        out_shape=jax.ShapeDtypeStruct((n, H), jnp.float32),
    )(S0, C0, S1, C1, x, W0T, W1T, WrT, b)


def _dense1(S0, C0, x, W0T, WrT, b):
    n = x.shape[0]
    grid = (n // _BLK,)
    sspec = pl.BlockSpec((2, _BLK, HH), lambda i: (0, i, 0))
    cspec = pl.BlockSpec((8, _BLK, 16), lambda i: (0, i, 0))
    wspec = pl.BlockSpec((H, H), lambda i: (0, 0))
    bspec = pl.BlockSpec((1, H), lambda i: (0, 0))
    xspec = pl.BlockSpec((_BLK, H), lambda i: (i, 0))
    return pl.pallas_call(
        _dense1_body,
        grid=grid,
        in_specs=[sspec, cspec, xspec, wspec, wspec, bspec],
        out_specs=xspec,
        out_shape=jax.ShapeDtypeStruct((n, H), jnp.float32),
    )(S0, C0, x, W0T, WrT, b)


def _pad_edges(ei, ndst, np_):
    pad_n = EP - E
    ar = jnp.arange(pad_n, dtype=jnp.int32)
    pad_src = ar % NJ
    pad_dst = ndst + (ar % (np_ - ndst))
    srcp = jnp.concatenate([ei[0], pad_src]).reshape(EP // ROW, ROW)
    dstp = jnp.concatenate([ei[1], pad_dst]).reshape(EP // ROW, ROW)
    return srcp, dstp


def kernel(x_job, x_station, x_machine, x_robot, ei_cbl, ei_li, ei_nd, ei_eb,
           ei_hb, Wl_cbl, bl_cbl, Wr_cbl, Wl_li, bl_li, Wr_li, Wl_nd, bl_nd,
           Wr_nd, Wl_eb, bl_eb, Wr_eb, Wl_hb, bl_hb, Wr_hb):
    x0h = x_job[:, :HH]
    x1h = x_job[:, HH:]
    eis = [_pad_edges(ei, ndst, np_)
           for ei, (ndst, np_) in zip([ei_cbl, ei_li, ei_nd, ei_eb, ei_hb],
                                      _RELS)]
    (S0, S1, S2, S3, S4), (C0, C1, C2, C3, C4) = _sc_segsums(x0h, x1h, eis)

    h_s = _dense2(S0, C0, S1, C1, x_station,
                  Wl_cbl.T, Wl_li.T, (Wr_cbl + Wr_li).T,
                  (bl_cbl + bl_li)[None, :])
    h_m = _dense2(S2, C2, S3, C3, x_machine,
                  Wl_nd.T, Wl_eb.T, (Wr_nd + Wr_eb).T,
                  (bl_nd + bl_eb)[None, :])
    h_r = _dense1(S4, C4, x_robot, Wl_hb.T, Wr_hb.T, bl_hb[None, :])
    return (h_s, h_m, h_r)


# speedup vs baseline: 1.3714x; 1.2430x over previous
import jax
import jax.numpy as jnp
from jax import lax
from jax.experimental import pallas as pl
from jax.experimental.pallas import tpu as pltpu
from jax.experimental.pallas import tpu_sc as plsc

H = 128
HH = 64
NJ = 50000
E = 500000
EP = 524288
ROW = 128
RPS = EP // (16 * ROW)
SB = 32
NSB = RPS // SB
NBUF = 4

_RELS = [(20000, 20096), (20000, 20096), (20000, 20096),
         (20000, 20096), (10000, 10112)]
_ACC_ROWS = 20096


def _sc_counts_body(dst0, dst1, dst2, dst3, dst4,
                    ones_h, z16,
                    C0, C1, C2, C3, C4,
                    didx, ones, cacc, csem):
    c = lax.axis_index("c")
    s = lax.axis_index("s")
    dsts = [dst0, dst1, dst2, dst3, dst4]
    Cs = [C0, C1, C2, C3, C4]
    pltpu.sync_copy(ones_h, ones)
    for r, (ndst, np_) in enumerate(_RELS):
        nz = np_ // 16
        pltpu.sync_copy(z16.at[pl.ds(0, nz)], cacc.at[pl.ds(s * nz, nz)])
        plsc.subcore_barrier()
        dst2d = dsts[r]

        @pl.loop(4 * c, 4 * c + 4)
        def _(b):
            pltpu.sync_copy(dst2d.at[pl.ds(s * RPS + b * SB, SB)],
                            didx.at[0])

            @pl.loop(0, SB)
            def _(j):
                pltpu.async_copy(ones, cacc.at[didx.at[0, j]],
                                 csem, add=True)
                pltpu.make_async_copy(
                    ones, cacc.at[didx.at[0, 0]], csem).wait()

        plsc.subcore_barrier()
        pltpu.sync_copy(cacc.at[pl.ds(s * nz, nz)],
                        Cs[r].at[c, pl.ds(s * nz, nz)])


def _sc_rows_body(x0h, x1h,
                  src0, dst0, src1, dst1, src2, dst2, src3, dst3, src4, dst4,
                  z64,
                  S0, S1, S2, S3, S4,
                  sidx, didx, r0, r1, r2, r3, acc,
                  isem0, isem1,
                  g0, g1, g2, g3, s0, s1, s2, s3):
    c = lax.axis_index("c")
    s = lax.axis_index("s")
    srcs = [src0, src1, src2, src3, src4]
    dsts = [dst0, dst1, dst2, dst3, dst4]
    Ss = [S0, S1, S2, S3, S4]
    rbufs = [r0, r1, r2, r3]
    gsems = [g0, g1, g2, g3]
    ssems = [s0, s1, s2, s3]
    isems = [isem0, isem1]

    for r, (ndst, np_) in enumerate(_RELS):
        nz = np_ // 16
        pltpu.sync_copy(z64.at[pl.ds(0, nz)], acc.at[pl.ds(s * nz, nz)])
        plsc.subcore_barrier()
        src2d, dst2d = srcs[r], dsts[r]

        def fire_stage(b, q):
            pltpu.async_copy(src2d.at[pl.ds(s * RPS + b * SB, SB)],
                             sidx.at[q], isems[q])
            pltpu.async_copy(dst2d.at[pl.ds(s * RPS + b * SB, SB)],
                             didx.at[q], isems[q])

        def wait_stage(q):
            pltpu.make_async_copy(src2d.at[pl.ds(s * RPS, SB)],
                                  sidx.at[q], isems[q]).wait()
            pltpu.make_async_copy(dst2d.at[pl.ds(s * RPS, SB)],
                                  didx.at[q], isems[q]).wait()

        def fire_gather(q, jrow, rbuf, sem):
            @pl.when(c == 0)
            def _():
                pltpu.async_copy(x0h.at[sidx.at[q, jrow]], rbuf, sem)

            @pl.when(c == 1)
            def _():
                pltpu.async_copy(x1h.at[sidx.at[q, jrow]], rbuf, sem)

        def wait_gather(rbuf, sem):
            pltpu.make_async_copy(x0h.at[sidx.at[0, 0]], rbuf, sem).wait()

        def fire_scatter(q, jrow, rbuf, sem):
            pltpu.async_copy(rbuf, acc.at[didx.at[q, jrow]], sem, add=True)

        def wait_scatter(rbuf, sem):
            pltpu.make_async_copy(rbuf, acc.at[didx.at[0, 0]], sem).wait()

        def run_rows(q):
            for k in range(NBUF - 1):
                fire_gather(q, k, rbufs[k], gsems[k])

            @pl.loop(0, SB // NBUF)
            def _(p):
                for t in range(NBUF):
                    j = NBUF * p + t
                    wait_gather(rbufs[t], gsems[t])
                    if t == 0:
                        @pl.when(p > 0)
                        def _():
                            wait_scatter(rbufs[t], ssems[0])
                    else:
                        wait_scatter(rbufs[t], ssems[0])
                    fire_scatter(q, j, rbufs[t], ssems[0])
                    nt = (t + NBUF - 1) % NBUF
                    lim = (SB - 1 - t - (NBUF - 1)) // NBUF

                    def adv():
                        fire_gather(q, j + NBUF - 1, rbufs[nt], gsems[nt])

                    if lim >= SB // NBUF - 1:
                        adv()
                    else:
                        @pl.when(p <= lim)
                        def _():
                            adv()

            wait_scatter(rbufs[0], ssems[0])

        fire_stage(0, 0)
        wait_stage(0)

        @pl.loop(0, NSB)
        def _(b):
            q = lax.rem(b, 2)

            @pl.when((b > 0) & (q == 0))
            def _():
                wait_stage(0)

            @pl.when((b > 0) & (q != 0))
            def _():
                wait_stage(1)

            @pl.when((b + 1 < NSB) & (q == 0))
            def _():
                fire_stage(b + 1, 1)

            @pl.when((b + 1 < NSB) & (q != 0))
            def _():
                fire_stage(b + 1, 0)

            run_rows(q)

        plsc.subcore_barrier()
        pltpu.sync_copy(acc.at[pl.ds(s * nz, nz)],
                        Ss[r].at[c, pl.ds(s * nz, nz)])


def _sc_segsums(x0h, x1h, eis):
    mesh = plsc.VectorSubcoreMesh(core_axis_name="c", subcore_axis_name="s")
    zrows = _ACC_ROWS // 16
    cnt_out = [jax.ShapeDtypeStruct((2, np_, 16), jnp.float32)
               for _, np_ in _RELS]
    row_out = [jax.ShapeDtypeStruct((2, np_, HH), jnp.float32)
               for _, np_ in _RELS]
    cnt_fn = pl.kernel(
        _sc_counts_body,
        out_type=cnt_out,
        mesh=mesh,
        scratch_types=[
            pltpu.VMEM((1, SB, ROW), jnp.int32),
            pltpu.VMEM((ROW, 16), jnp.float32),
            pltpu.VMEM_SHARED((_ACC_ROWS, 16), jnp.float32),
            pltpu.SemaphoreType.DMA,
        ],
        compiler_params=pltpu.CompilerParams(use_tc_tiling_on_sc=False),
    )
    row_fn = pl.kernel(
        _sc_rows_body,
        out_type=row_out,
        mesh=mesh,
        scratch_types=[
            pltpu.VMEM((2, SB, ROW), jnp.int32),
            pltpu.VMEM((2, SB, ROW), jnp.int32),
            pltpu.VMEM((ROW, HH), jnp.float32),
            pltpu.VMEM((ROW, HH), jnp.float32),
            pltpu.VMEM((ROW, HH), jnp.float32),
            pltpu.VMEM((ROW, HH), jnp.float32),
            pltpu.VMEM_SHARED((_ACC_ROWS, HH), jnp.float32),
            pltpu.SemaphoreType.DMA,
            pltpu.SemaphoreType.DMA,
            pltpu.SemaphoreType.DMA,
            pltpu.SemaphoreType.DMA,
            pltpu.SemaphoreType.DMA,
            pltpu.SemaphoreType.DMA,
            pltpu.SemaphoreType.DMA,
            pltpu.SemaphoreType.DMA,
            pltpu.SemaphoreType.DMA,
            pltpu.SemaphoreType.DMA,
        ],
        compiler_params=pltpu.CompilerParams(use_tc_tiling_on_sc=False),
    )
    ones_h = jnp.ones((ROW, 16), jnp.float32)
    z64 = jnp.zeros((zrows, HH), jnp.float32)
    z16 = jnp.zeros((zrows, 16), jnp.float32)
    dst_flat = [dstp for (_, dstp) in eis]
    sd_flat = []
    for (srcp, dstp) in eis:
        sd_flat.append(srcp)
        sd_flat.append(dstp)
    Cs = cnt_fn(*dst_flat, ones_h, z16)
    Ss = row_fn(x0h, x1h, *sd_flat, z64)
    return Ss, Cs


def _dense2_body(S0, C0, S1, C1, x, W0T, W1T, WrT, b, o):
    c0 = jnp.maximum(jnp.sum(C0[:, :, 0:1], axis=0), 1.0)
    c1 = jnp.maximum(jnp.sum(C1[:, :, 0:1], axis=0), 1.0)
    m0 = jnp.concatenate([S0[0], S0[1]], axis=1) / c0
    m1 = jnp.concatenate([S1[0], S1[1]], axis=1) / c1
    acc = (jnp.dot(m0, W0T[...], preferred_element_type=jnp.float32,
                   precision=lax.Precision.HIGHEST)
           + jnp.dot(m1, W1T[...], preferred_element_type=jnp.float32,
                     precision=lax.Precision.HIGHEST)
           + jnp.dot(x[...], WrT[...], preferred_element_type=jnp.float32,
                     precision=lax.Precision.HIGHEST)
           + b[...])
    o[...] = jnp.maximum(acc, 0.0)


def _dense1_body(S0, C0, x, W0T, WrT, b, o):
    c0 = jnp.maximum(jnp.sum(C0[:, :, 0:1], axis=0), 1.0)
    m0 = jnp.concatenate([S0[0], S0[1]], axis=1) / c0
    acc = (jnp.dot(m0, W0T[...], preferred_element_type=jnp.float32,
                   precision=lax.Precision.HIGHEST)
           + jnp.dot(x[...], WrT[...], preferred_element_type=jnp.float32,
                     precision=lax.Precision.HIGHEST)
           + b[...])
    o[...] = jnp.maximum(acc, 0.0)


_BLK = 2000


def _dense2(S0, C0, S1, C1, x, W0T, W1T, WrT, b):
    n = x.shape[0]
    grid = (n // _BLK,)
    sspec = pl.BlockSpec((2, _BLK, HH), lambda i: (0, i, 0))
    cspec = pl.BlockSpec((2, _BLK, 16), lambda i: (0, i, 0))
    wspec = pl.BlockSpec((H, H), lambda i: (0, 0))
    bspec = pl.BlockSpec((1, H), lambda i: (0, 0))
    xspec = pl.BlockSpec((_BLK, H), lambda i: (i, 0))
    return pl.pallas_call(
        _dense2_body,
        grid=grid,
        in_specs=[sspec, cspec, sspec, cspec, xspec, wspec, wspec, wspec, bspec],
        out_specs=xspec,
        out_shape=jax.ShapeDtypeStruct((n, H), jnp.float32),
    )(S0, C0, S1, C1, x, W0T, W1T, WrT, b)


def _dense1(S0, C0, x, W0T, WrT, b):
    n = x.shape[0]
    grid = (n // _BLK,)
    sspec = pl.BlockSpec((2, _BLK, HH), lambda i: (0, i, 0))
    cspec = pl.BlockSpec((2, _BLK, 16), lambda i: (0, i, 0))
    wspec = pl.BlockSpec((H, H), lambda i: (0, 0))
    bspec = pl.BlockSpec((1, H), lambda i: (0, 0))
    xspec = pl.BlockSpec((_BLK, H), lambda i: (i, 0))
    return pl.pallas_call(
        _dense1_body,
        grid=grid,
        in_specs=[sspec, cspec, xspec, wspec, wspec, bspec],
        out_specs=xspec,
        out_shape=jax.ShapeDtypeStruct((n, H), jnp.float32),
    )(S0, C0, x, W0T, WrT, b)


def _pad_edges(ei, ndst, np_):
    pad_n = EP - E
    ar = jnp.arange(pad_n, dtype=jnp.int32)
    pad_src = ar % NJ
    pad_dst = ndst + (ar % (np_ - ndst))
    srcp = jnp.concatenate([ei[0], pad_src]).reshape(EP // ROW, ROW)
    dstp = jnp.concatenate([ei[1], pad_dst]).reshape(EP // ROW, ROW)
    return srcp, dstp


def kernel(x_job, x_station, x_machine, x_robot, ei_cbl, ei_li, ei_nd, ei_eb,
           ei_hb, Wl_cbl, bl_cbl, Wr_cbl, Wl_li, bl_li, Wr_li, Wl_nd, bl_nd,
           Wr_nd, Wl_eb, bl_eb, Wr_eb, Wl_hb, bl_hb, Wr_hb):
    x0h = x_job[:, :HH]
    x1h = x_job[:, HH:]
    eis = [_pad_edges(ei, ndst, np_)
           for ei, (ndst, np_) in zip([ei_cbl, ei_li, ei_nd, ei_eb, ei_hb],
                                      _RELS)]
    (S0, S1, S2, S3, S4), (C0, C1, C2, C3, C4) = _sc_segsums(x0h, x1h, eis)

    h_s = _dense2(S0, C0, S1, C1, x_station,
                  Wl_cbl.T, Wl_li.T, (Wr_cbl + Wr_li).T,
                  (bl_cbl + bl_li)[None, :])
    h_m = _dense2(S2, C2, S3, C3, x_machine,
                  Wl_nd.T, Wl_eb.T, (Wr_nd + Wr_eb).T,
                  (bl_nd + bl_eb)[None, :])
    h_r = _dense1(S4, C4, x_robot, Wl_hb.T, Wr_hb.T, bl_hb[None, :])
    return (h_s, h_m, h_r)
